# baseline (speedup 1.0000x reference)
import functools

import jax
import jax.numpy as jnp
from jax import lax
from jax.experimental import pallas as pl
from jax.experimental.pallas import tpu as pltpu
from jax.experimental.pallas import tpu_sc as plsc

VOCAB = 1000
B, S = 1024, 50
ROWS = B * S
NC, NS, L = 2, 16, 16
NW = NC * NS
ROWS_PER_W = ROWS // NW
CHUNK = 64
NCHUNK = ROWS_PER_W // CHUNK
CHUNK_WORDS = CHUNK * VOCAB
ZERO_UNROLL = 8


def _sc_onehot(pos):
    mesh = plsc.VectorSubcoreMesh(core_axis_name="c", subcore_axis_name="s")

    @functools.partial(
        pl.kernel,
        mesh=mesh,
        out_type=jax.ShapeDtypeStruct((ROWS * VOCAB,), jnp.float32),
        scratch_types=[
            pltpu.VMEM((ROWS_PER_W,), jnp.int32),
            pltpu.VMEM((CHUNK_WORDS,), jnp.float32),
        ],
        compiler_params=pltpu.CompilerParams(needs_layout_passes=False),
    )
    def k(pos_hbm, out_hbm, pos_v, buf_v):
        wid = lax.axis_index("s") * NC + lax.axis_index("c")
        base_row = wid * ROWS_PER_W
        pltpu.sync_copy(pos_hbm.at[pl.ds(base_row, ROWS_PER_W)], pos_v)

        zeros16 = jnp.zeros((L,), jnp.float32)
        ones16 = jnp.ones((L,), jnp.float32)
        lane = lax.iota(jnp.int32, L)

        def zero_body(i, c):
            for u in range(ZERO_UNROLL):
                buf_v[pl.ds((i * ZERO_UNROLL + u) * L, L)] = zeros16
            return c

        lax.fori_loop(0, CHUNK_WORDS // (L * ZERO_UNROLL), zero_body, 0)

        def chunk_body(t, c):
            for g in range(CHUNK // L):
                p16 = pos_v[pl.ds(t * CHUNK + g * L, L)]
                idx = (lane + g * L) * VOCAB + p16
                plsc.store_scatter(buf_v, [idx], ones16)
            pltpu.sync_copy(
                buf_v,
                out_hbm.at[pl.ds((base_row + t * CHUNK) * VOCAB, CHUNK_WORDS)],
            )
            for g in range(CHUNK // L):
                p16 = pos_v[pl.ds(t * CHUNK + g * L, L)]
                idx = (lane + g * L) * VOCAB + p16
                plsc.store_scatter(buf_v, [idx], zeros16)
            return c

        lax.fori_loop(0, NCHUNK, chunk_body, 0)

    return k(pos)


def kernel(x):
    del x
    pos = jax.random.randint(
        jax.random.key(42), (B, S), 1, 6, dtype=jnp.int32
    ).reshape(-1)
    out = _sc_onehot(pos)
    return out.reshape(B, S, VOCAB)
